# baseline (speedup 1.0000x reference)
import functools

import jax
import jax.numpy as jnp
import numpy as np
from jax import lax
from jax.experimental import pallas as pl
from jax.experimental.pallas import tpu as pltpu
from jax.experimental.pallas import tpu_sc as plsc

_PCM = np.array([
    [1, 0, 1, 0, 1, 0, 1, 0, 1, 0, 1, 0, 1, 0, 1, 0, 1, 0, 1, 0, 1, 0, 1, 0, 1, 0, 1, 0, 1, 0, 1],
    [0, 1, 1, 0, 0, 1, 1, 0, 0, 1, 1, 0, 0, 1, 1, 0, 0, 1, 1, 0, 0, 1, 1, 0, 0, 1, 1, 0, 0, 1, 1],
    [0, 0, 0, 1, 1, 1, 1, 0, 0, 0, 0, 1, 1, 1, 1, 0, 0, 0, 0, 1, 1, 1, 1, 0, 0, 0, 0, 1, 1, 1, 1],
    [0, 0, 0, 0, 0, 0, 0, 1, 1, 1, 1, 1, 1, 1, 1, 0, 0, 0, 0, 0, 0, 0, 0, 1, 1, 1, 1, 1, 1, 1, 1],
    [0, 0, 0, 0, 0, 0, 0, 0, 0, 0, 0, 0, 0, 0, 0, 1, 1, 1, 1, 1, 1, 1, 1, 1, 1, 1, 1, 1, 1, 1, 1],
], dtype=np.int64)
_ROLLED = np.stack(np.where(_PCM), axis=1)
_NCHK, _NVAR = _PCM.shape
_E = _ROLLED.shape[0]
_DEG = 16
_COLS = _ROLLED[:, 1].reshape(_NCHK, _DEG)
_COL_EDGES = [np.where(_ROLLED[:, 1] == v)[0].tolist() for v in range(_NVAR)]
_NUM_ITER = 20

_SQRT2 = 1.4142135
_LN2_HI = 0.69314575
_LN2_LO = 1.4286068e-06
_INVLN2 = 1.4426950408889634
_LOGC = [-6.900793061981325e-10, 0.9999999966211686, -0.4999996532905042,
         0.3333335964540943, -0.25002812462237173, 0.2000147231976227,
         -0.16586843259976403, 0.14176370184103376, -0.13388417444788703,
         0.12990627398860688, -0.07417228391986355]
_EXPC = [0.9999999999595321, 1.000000037739721, 0.5000000107781664,
         0.16666415422747397, 0.04166621818498068, 0.008375133426734613,
         0.0013948586767683234]
_RCPC = [2.8499173034659373, -2.9844526530961426, 1.3616308933192882,
         -0.22857251654217783]


def _horner(coefs, x):
    acc = jnp.full(x.shape, coefs[-1], x.dtype)
    for c in coefs[-2::-1]:
        acc = acc * x + c
    return acc


_RLO = float(1e-7 / (2.0 - 1e-7))
_RHI = float((2.0 - 1e-7) / 1e-7)


def _sc_log(e):
    i = plsc.bitcast(e, jnp.int32)
    k = (i >> 23) - 127
    m = plsc.bitcast((i & 0x7FFFFF) | 0x3F800000, jnp.float32)
    big = m > _SQRT2
    m = jnp.where(big, m * 0.5, m)
    kf = (k + jnp.where(big, 1, 0)).astype(jnp.float32)
    p = _horner(_LOGC, m - 1.0)
    return kf * _LN2_HI + (kf * _LN2_LO + p)


def _sc_atanh2(x):
    r = jnp.clip((1.0 + x) / (1.0 - x), _RLO, _RHI)
    return _sc_log(r)


def _sc_tanh12(y):
    t = jnp.exp(-jnp.abs(y))
    q = (1.0 - t) / (1.0 + t)
    return jnp.where(y < 0.0, -q, q)


def _loo_products(grp):
    n = len(grp)
    pref = [grp[0]]
    for k in range(1, n):
        pref.append(pref[-1] * grp[k])
    suf = [grp[n - 1]]
    for k in range(n - 2, -1, -1):
        suf.append(suf[-1] * grp[k])
    suf = suf[::-1]
    out = []
    for k in range(n):
        if k == 0:
            out.append(suf[1])
        elif k == n - 1:
            out.append(pref[n - 2])
        else:
            out.append(pref[k - 1] * suf[k + 1])
    return out


_NW = 32
_B_SC = 16 * _NW


def _sc_bp_body(llr_hbm, out_hbm, llr16_v, llr_v, msg_v, he_v, out16_v):
    wid = lax.axis_index("s") * 2 + lax.axis_index("c")
    pltpu.sync_copy(llr_hbm.at[pl.ds(wid * 16, 16)], llr16_v)
    rows = lax.iota(jnp.int32, 16)
    for v in range(_NVAR):
        col = jnp.full((16,), v, jnp.int32)
        t = plsc.load_gather(llr16_v, [rows, col])
        llr_v[v, :] = t
        t = _sc_tanh12(t)
        for e in _COL_EDGES[v]:
            msg_v[e, :] = t

    def it_body(it, c2):
        cs = [None] * _NVAR
        for c in range(_NCHK):
            grp = [msg_v[c * _DEG + k, :] for k in range(_DEG)]
            loo = _loo_products(grp)
            for k in range(_DEG):
                he = _sc_atanh2(loo[k])
                e = c * _DEG + k
                he_v[e, :] = he
                v = int(_COLS[c, k])
                cs[v] = he if cs[v] is None else cs[v] + he
        for v in range(_NVAR):
            base = cs[v] + llr_v[v, :]
            col = jnp.full((16,), v, jnp.int32)
            plsc.store_scatter(out16_v, [rows, col], base)
            for e in _COL_EDGES[v]:
                msg_v[e, :] = _sc_tanh12(base - he_v[e, :])
        return c2

    lax.fori_loop(0, _NUM_ITER, it_body, 0)
    pltpu.sync_copy(out16_v, out_hbm.at[pl.ds(wid * 16, 16)])


@jax.jit
def _sc_bp(llr_head):
    mesh = plsc.VectorSubcoreMesh(
        core_axis_name="c", subcore_axis_name="s", num_cores=2, num_subcores=16)
    return pl.kernel(
        _sc_bp_body,
        out_type=jax.ShapeDtypeStruct((_B_SC, _NVAR), jnp.float32),
        mesh=mesh,
        compiler_params=pltpu.CompilerParams(needs_layout_passes=False),
        scratch_types=[
            pltpu.VMEM((16, _NVAR), jnp.float32),
            pltpu.VMEM((_NVAR, 16), jnp.float32),
            pltpu.VMEM((_E, 16), jnp.float32),
            pltpu.VMEM((_E, 16), jnp.float32),
            pltpu.VMEM((16, _NVAR), jnp.float32),
        ],
    )(llr_head)


def _tc_bp_block(llr_rows, s):
    h_r = [llr_rows[int(_COLS[c, k])] for c in range(_NCHK) for k in range(_DEG)]

    def body(_, carry):
        m_stack, _cs = carry
        msg = [m_stack[e * s:(e + 1) * s] for e in range(_E)]
        h_e = [None] * _E
        cs = [None] * _NVAR
        for c in range(_NCHK):
            loo = _loo_products(msg[c * _DEG:(c + 1) * _DEG])
            for k in range(_DEG):
                e1 = jnp.clip(1.0 + loo[k], 1e-07, 2.0 - 1e-07)
                e2 = jnp.clip(1.0 - loo[k], 1e-07, 2.0 - 1e-07)
                he = jnp.log(e1 / e2)
                e = c * _DEG + k
                h_e[e] = he
                v = int(_COLS[c, k])
                cs[v] = he if cs[v] is None else cs[v] + he
        new_msg = [
            jnp.tanh((cs[int(_COLS[c, k])] - h_e[c * _DEG + k]
                      + h_r[c * _DEG + k]) * 0.5)
            for c in range(_NCHK) for k in range(_DEG)
        ]
        return jnp.concatenate(new_msg, axis=0), jnp.concatenate(cs, axis=0)

    msg0 = [jnp.tanh(h * 0.5) for h in h_r]
    cs0 = jnp.zeros((_NVAR * s, llr_rows[0].shape[1]), jnp.float32)
    _, cs_fin = jax.lax.fori_loop(
        0, _NUM_ITER, body, (jnp.concatenate(msg0, axis=0), cs0))
    out = [cs_fin[v * s:(v + 1) * s] + llr_rows[v] for v in range(_NVAR)]
    return jnp.concatenate(out, axis=0)


def _tc_kernel_body(llr_ref, out_ref, *, s):
    llr_rows = [llr_ref[v * s:(v + 1) * s] for v in range(_NVAR)]
    out_ref[...] = _tc_bp_block(llr_rows, s)


def _tc_bp(llr_part, grid):
    Bt = llr_part.shape[0]
    S = 8
    W = Bt // S
    WT = W // grid
    llr2 = llr_part.T.reshape(_NVAR * S, W)
    out2 = pl.pallas_call(
        functools.partial(_tc_kernel_body, s=S),
        grid=(grid,),
        in_specs=[pl.BlockSpec((_NVAR * S, WT), lambda i: (0, i))],
        out_specs=pl.BlockSpec((_NVAR * S, WT), lambda i: (0, i)),
        out_shape=jax.ShapeDtypeStruct((_NVAR * S, W), jnp.float32),
    )(llr2)
    return out2.reshape(_NVAR, Bt).T


@jax.jit
def kernel(llr):
    out_sc = _sc_bp(llr)
    out_tc = _tc_bp(llr, grid=4)
    return jnp.concatenate([out_sc, out_tc[_B_SC:]], axis=0)


# speedup vs baseline: 1.0141x; 1.0141x over previous
import functools

import jax
import jax.numpy as jnp
import numpy as np
from jax import lax
from jax.experimental import pallas as pl
from jax.experimental.pallas import tpu as pltpu
from jax.experimental.pallas import tpu_sc as plsc

_PCM = np.array([
    [1, 0, 1, 0, 1, 0, 1, 0, 1, 0, 1, 0, 1, 0, 1, 0, 1, 0, 1, 0, 1, 0, 1, 0, 1, 0, 1, 0, 1, 0, 1],
    [0, 1, 1, 0, 0, 1, 1, 0, 0, 1, 1, 0, 0, 1, 1, 0, 0, 1, 1, 0, 0, 1, 1, 0, 0, 1, 1, 0, 0, 1, 1],
    [0, 0, 0, 1, 1, 1, 1, 0, 0, 0, 0, 1, 1, 1, 1, 0, 0, 0, 0, 1, 1, 1, 1, 0, 0, 0, 0, 1, 1, 1, 1],
    [0, 0, 0, 0, 0, 0, 0, 1, 1, 1, 1, 1, 1, 1, 1, 0, 0, 0, 0, 0, 0, 0, 0, 1, 1, 1, 1, 1, 1, 1, 1],
    [0, 0, 0, 0, 0, 0, 0, 0, 0, 0, 0, 0, 0, 0, 0, 1, 1, 1, 1, 1, 1, 1, 1, 1, 1, 1, 1, 1, 1, 1, 1],
], dtype=np.int64)
_ROLLED = np.stack(np.where(_PCM), axis=1)
_NCHK, _NVAR = _PCM.shape
_E = _ROLLED.shape[0]
_DEG = 16
_COLS = _ROLLED[:, 1].reshape(_NCHK, _DEG)
_COL_EDGES = [np.where(_ROLLED[:, 1] == v)[0].tolist() for v in range(_NVAR)]
_NUM_ITER = 20

_SQRT2 = 1.4142135
_LN2_HI = 0.69314575
_LN2_LO = 1.4286068e-06
_INVLN2 = 1.4426950408889634
_LOGC = [-6.900793061981325e-10, 0.9999999966211686, -0.4999996532905042,
         0.3333335964540943, -0.25002812462237173, 0.2000147231976227,
         -0.16586843259976403, 0.14176370184103376, -0.13388417444788703,
         0.12990627398860688, -0.07417228391986355]
_EXPC = [0.9999999999595321, 1.000000037739721, 0.5000000107781664,
         0.16666415422747397, 0.04166621818498068, 0.008375133426734613,
         0.0013948586767683234]
_RCPC = [2.8499173034659373, -2.9844526530961426, 1.3616308933192882,
         -0.22857251654217783]


def _horner(coefs, x):
    acc = jnp.full(x.shape, coefs[-1], x.dtype)
    for c in coefs[-2::-1]:
        acc = acc * x + c
    return acc


_RLO = float(1e-7 / (2.0 - 1e-7))
_RHI = float((2.0 - 1e-7) / 1e-7)


def _sc_log(e):
    i = plsc.bitcast(e, jnp.int32)
    k = (i >> 23) - 127
    m = plsc.bitcast((i & 0x7FFFFF) | 0x3F800000, jnp.float32)
    big = m > _SQRT2
    m = jnp.where(big, m * 0.5, m)
    kf = (k + jnp.where(big, 1, 0)).astype(jnp.float32)
    p = _horner(_LOGC, m - 1.0)
    return kf * _LN2_HI + (kf * _LN2_LO + p)


def _sc_atanh2(x):
    r = jnp.clip((1.0 + x) / (1.0 - x), _RLO, _RHI)
    return _sc_log(r)


def _sc_tanh12(y):
    t = jnp.exp(-jnp.abs(y))
    q = (1.0 - t) / (1.0 + t) + 1e-30
    return jnp.where(y < 0.0, -q, q)


def _loo_products(grp):
    n = len(grp)
    pref = [grp[0]]
    for k in range(1, n):
        pref.append(pref[-1] * grp[k])
    suf = [grp[n - 1]]
    for k in range(n - 2, -1, -1):
        suf.append(suf[-1] * grp[k])
    suf = suf[::-1]
    out = []
    for k in range(n):
        if k == 0:
            out.append(suf[1])
        elif k == n - 1:
            out.append(pref[n - 2])
        else:
            out.append(pref[k - 1] * suf[k + 1])
    return out


_NW = 32
_B_SC = 16 * _NW


def _sc_bp_body(llr_hbm, out_hbm, llr16_v, llr_v, msg_v, out16_v):
    wid = lax.axis_index("s") * 2 + lax.axis_index("c")
    pltpu.sync_copy(llr_hbm.at[pl.ds(wid * 16, 16)], llr16_v)
    rows = lax.iota(jnp.int32, 16)
    for v in range(_NVAR):
        col = jnp.full((16,), v, jnp.int32)
        t = plsc.load_gather(llr16_v, [rows, col])
        llr_v[v, :] = t
        t = _sc_tanh12(t)
        for e in _COL_EDGES[v]:
            msg_v[e, :] = t

    def it_body(it, c2):
        cs = [None] * _NVAR
        h_e = [None] * _E
        for c in range(_NCHK):
            grp = [msg_v[c * _DEG + k, :] for k in range(_DEG)]
            P = grp[0]
            for k in range(1, _DEG):
                P = P * grp[k]
            for k in range(_DEG):
                m = grp[k]
                r = jnp.clip((m + P) / (m - P), _RLO, _RHI)
                he = _sc_log(r)
                e = c * _DEG + k
                h_e[e] = he
                v = int(_COLS[c, k])
                cs[v] = he if cs[v] is None else cs[v] + he
        for v in range(_NVAR):
            base = cs[v] + llr_v[v, :]
            for e in _COL_EDGES[v]:
                msg_v[e, :] = _sc_tanh12(base - h_e[e])

            @pl.when(it == _NUM_ITER - 1)
            def _():
                col = jnp.full((16,), v, jnp.int32)
                plsc.store_scatter(out16_v, [rows, col], base)
        return c2

    lax.fori_loop(0, _NUM_ITER, it_body, 0)
    pltpu.sync_copy(out16_v, out_hbm.at[pl.ds(wid * 16, 16)])


@jax.jit
def _sc_bp(llr_head):
    mesh = plsc.VectorSubcoreMesh(
        core_axis_name="c", subcore_axis_name="s", num_cores=2, num_subcores=16)
    return pl.kernel(
        _sc_bp_body,
        out_type=jax.ShapeDtypeStruct((_B_SC, _NVAR), jnp.float32),
        mesh=mesh,
        compiler_params=pltpu.CompilerParams(needs_layout_passes=False),
        scratch_types=[
            pltpu.VMEM((16, _NVAR), jnp.float32),
            pltpu.VMEM((_NVAR, 16), jnp.float32),
            pltpu.VMEM((_E, 16), jnp.float32),
            pltpu.VMEM((16, _NVAR), jnp.float32),
        ],
    )(llr_head)


def _tc_bp_block(llr_rows, s):
    h_r = [llr_rows[int(_COLS[c, k])] for c in range(_NCHK) for k in range(_DEG)]

    def body(_, carry):
        m_stack, _cs = carry
        msg = [m_stack[e * s:(e + 1) * s] for e in range(_E)]
        h_e = [None] * _E
        cs = [None] * _NVAR
        for c in range(_NCHK):
            loo = _loo_products(msg[c * _DEG:(c + 1) * _DEG])
            for k in range(_DEG):
                e1 = jnp.clip(1.0 + loo[k], 1e-07, 2.0 - 1e-07)
                e2 = jnp.clip(1.0 - loo[k], 1e-07, 2.0 - 1e-07)
                he = jnp.log(e1 / e2)
                e = c * _DEG + k
                h_e[e] = he
                v = int(_COLS[c, k])
                cs[v] = he if cs[v] is None else cs[v] + he
        new_msg = [
            jnp.tanh((cs[int(_COLS[c, k])] - h_e[c * _DEG + k]
                      + h_r[c * _DEG + k]) * 0.5)
            for c in range(_NCHK) for k in range(_DEG)
        ]
        return jnp.concatenate(new_msg, axis=0), jnp.concatenate(cs, axis=0)

    msg0 = [jnp.tanh(h * 0.5) for h in h_r]
    cs0 = jnp.zeros((_NVAR * s, llr_rows[0].shape[1]), jnp.float32)
    _, cs_fin = jax.lax.fori_loop(
        0, _NUM_ITER, body, (jnp.concatenate(msg0, axis=0), cs0))
    out = [cs_fin[v * s:(v + 1) * s] + llr_rows[v] for v in range(_NVAR)]
    return jnp.concatenate(out, axis=0)


def _tc_kernel_body(llr_ref, out_ref, *, s):
    llr_rows = [llr_ref[v * s:(v + 1) * s] for v in range(_NVAR)]
    out_ref[...] = _tc_bp_block(llr_rows, s)


def _tc_bp(llr_part, grid):
    Bt = llr_part.shape[0]
    S = 8
    W = Bt // S
    WT = W // grid
    llr2 = llr_part.T.reshape(_NVAR * S, W)
    out2 = pl.pallas_call(
        functools.partial(_tc_kernel_body, s=S),
        grid=(grid,),
        in_specs=[pl.BlockSpec((_NVAR * S, WT), lambda i: (0, i))],
        out_specs=pl.BlockSpec((_NVAR * S, WT), lambda i: (0, i)),
        out_shape=jax.ShapeDtypeStruct((_NVAR * S, W), jnp.float32),
    )(llr2)
    return out2.reshape(_NVAR, Bt).T


@jax.jit
def kernel(llr):
    out_sc = _sc_bp(llr)
    out_tc = _tc_bp(llr, grid=4)
    return jnp.concatenate([out_sc, out_tc[_B_SC:]], axis=0)
